# Initial kernel scaffold; baseline (speedup 1.0000x reference)
import jax
import jax.numpy as jnp
from jax.experimental import pallas as pl


def kernel(input_ids, params):
    raise NotImplementedError("write your pallas kernel here")


# speedup vs baseline: 1.7168x; 1.7168x over previous
import functools

import jax
import jax.numpy as jnp
from jax.experimental import pallas as pl
from jax.experimental.pallas import tpu as pltpu

S, D = 2048, 1024
H, KV, HD = 16, 8, 64
E, TOPK, F = 8, 2, 1024
L = 2
EPS = 1e-6
THETA = 10000.0
HH = HD // 2

QW = H * HH
KW = KV * HH


def _rms(x, w):
    return x * jax.lax.rsqrt(jnp.mean(x * x, axis=-1, keepdims=True) + EPS) * w


def _qkv_body(h_ref, ln1_ref, wq_ref, wk_ref, wv_ref, cq_ref, sq_ref,
              ck_ref, sk_ref, q_out, k_out, v_out):
    r = _rms(h_ref[...], ln1_ref[...])
    q = jnp.dot(r, wq_ref[...], preferred_element_type=jnp.float32)
    k = jnp.dot(r, wk_ref[...], preferred_element_type=jnp.float32)
    v = jnp.dot(r, wv_ref[...], preferred_element_type=jnp.float32)
    qr = jnp.concatenate([-q[:, QW:], q[:, :QW]], axis=1)
    kr = jnp.concatenate([-k[:, KW:], k[:, :KW]], axis=1)
    q_out[...] = q * cq_ref[...] + qr * sq_ref[...]
    k_out[...] = k * ck_ref[...] + kr * sk_ref[...]
    v_out[...] = v


BSQ = 512
NSB = S // BSQ


def _qkv_call(h, ln1, wq_p, wk_p, wv, cq, sq, ck, sk):
    return pl.pallas_call(
        _qkv_body,
        grid=(NSB,),
        in_specs=[
            pl.BlockSpec((BSQ, D), lambda i: (i, 0)),
            pl.BlockSpec((D,), lambda i: (0,)),
            pl.BlockSpec((D, 2 * QW), lambda i: (0, 0)),
            pl.BlockSpec((D, 2 * KW), lambda i: (0, 0)),
            pl.BlockSpec((D, KV * HD), lambda i: (0, 0)),
            pl.BlockSpec((BSQ, 2 * QW), lambda i: (i, 0)),
            pl.BlockSpec((BSQ, 2 * QW), lambda i: (i, 0)),
            pl.BlockSpec((BSQ, 2 * KW), lambda i: (i, 0)),
            pl.BlockSpec((BSQ, 2 * KW), lambda i: (i, 0)),
        ],
        out_specs=[
            pl.BlockSpec((BSQ, 2 * QW), lambda i: (i, 0)),
            pl.BlockSpec((BSQ, 2 * KW), lambda i: (i, 0)),
            pl.BlockSpec((BSQ, KV * HD), lambda i: (i, 0)),
        ],
        out_shape=[
            jax.ShapeDtypeStruct((S, 2 * QW), jnp.float32),
            jax.ShapeDtypeStruct((S, 2 * KW), jnp.float32),
            jax.ShapeDtypeStruct((S, KV * HD), jnp.float32),
        ],
    )(h, ln1, wq_p, wk_p, wv, cq, sq, ck, sk)


BQ = 256
NQB = S // BQ


def _attn_body(q_ref, k_ref, v_ref, o_ref):
    qb_i = pl.program_id(0)
    q = q_ref[...]
    k = k_ref[...]
    v = v_ref[...]
    rows = jax.lax.broadcasted_iota(jnp.int32, (BQ, S), 0) + qb_i * BQ
    cols = jax.lax.broadcasted_iota(jnp.int32, (BQ, S), 1)
    bias = jnp.where(cols <= rows, 0.0, -1e9)
    for h in range(H):
        j = h // 2
        qh = jnp.concatenate(
            [q[:, h * HH:(h + 1) * HH], q[:, QW + h * HH:QW + (h + 1) * HH]],
            axis=1)
        kh = jnp.concatenate(
            [k[:, j * HH:(j + 1) * HH], k[:, KW + j * HH:KW + (j + 1) * HH]],
            axis=1)
        s = jnp.dot(qh, kh.T, preferred_element_type=jnp.float32)
        s = s * (1.0 / (HD ** 0.5)) + bias
        p = jax.nn.softmax(s, axis=-1)
        o_ref[:, h * HD:(h + 1) * HD] = jnp.dot(
            p, v[:, j * HD:(j + 1) * HD], preferred_element_type=jnp.float32)


def _attn_call(q, k, v):
    return pl.pallas_call(
        _attn_body,
        grid=(NQB,),
        in_specs=[
            pl.BlockSpec((BQ, 2 * QW), lambda qb: (qb, 0)),
            pl.BlockSpec((S, 2 * KW), lambda qb: (0, 0)),
            pl.BlockSpec((S, KV * HD), lambda qb: (0, 0)),
        ],
        out_specs=pl.BlockSpec((BQ, H * HD), lambda qb: (qb, 0)),
        out_shape=jax.ShapeDtypeStruct((S, H * HD), jnp.float32),
        compiler_params=pltpu.CompilerParams(
            vmem_limit_bytes=100 * 1024 * 1024),
    )(q, k, v)


def _wo_router_body(h_ref, o_ref, wo_ref, ln2_ref, wg_ref,
                    h2_out, r2_out, gates_out):
    h2 = h_ref[...] + jnp.dot(o_ref[...], wo_ref[...],
                              preferred_element_type=jnp.float32)
    h2_out[...] = h2
    r2 = _rms(h2, ln2_ref[...])
    r2_out[...] = r2
    logits = jnp.dot(r2, wg_ref[...], preferred_element_type=jnp.float32)
    probs = jax.nn.softmax(logits, axis=-1)
    idx = jax.lax.broadcasted_iota(jnp.int32, (BSQ, E), 1)
    m1 = jnp.max(probs, axis=-1, keepdims=True)
    i1 = jnp.min(jnp.where(probs == m1, idx, E), axis=-1, keepdims=True)
    oh1 = idx == i1
    rest = jnp.where(oh1, -jnp.inf, probs)
    m2 = jnp.max(rest, axis=-1, keepdims=True)
    i2 = jnp.min(jnp.where(rest == m2, idx, E), axis=-1, keepdims=True)
    oh2 = idx == i2
    denom = m1 + m2
    gates_out[...] = (jnp.where(oh1, m1, 0.0)
                      + jnp.where(oh2, m2, 0.0)) / denom


def _wo_router_call(h, o, wo, ln2, wg):
    return pl.pallas_call(
        _wo_router_body,
        grid=(NSB,),
        in_specs=[
            pl.BlockSpec((BSQ, D), lambda i: (i, 0)),
            pl.BlockSpec((BSQ, H * HD), lambda i: (i, 0)),
            pl.BlockSpec((H * HD, D), lambda i: (0, 0)),
            pl.BlockSpec((D,), lambda i: (0,)),
            pl.BlockSpec((D, E), lambda i: (0, 0)),
        ],
        out_specs=[
            pl.BlockSpec((BSQ, D), lambda i: (i, 0)),
            pl.BlockSpec((BSQ, D), lambda i: (i, 0)),
            pl.BlockSpec((BSQ, E), lambda i: (i, 0)),
        ],
        out_shape=[
            jax.ShapeDtypeStruct((S, D), jnp.float32),
            jax.ShapeDtypeStruct((S, D), jnp.float32),
            jax.ShapeDtypeStruct((S, E), jnp.float32),
        ],
    )(h, o, wo, ln2, wg)


MOE_CHUNK = 512


def _moe_body(r2_ref, gates_ref, h2_ref, w1_ref, w3_ref, w2_ref, out_ref):
    e = pl.program_id(0)
    lane = jax.lax.broadcasted_iota(jnp.int32, (S, E), 1)
    g = jnp.sum(jnp.where(lane == e, gates_ref[...], 0.0),
                axis=-1, keepdims=True)
    for c in range(S // MOE_CHUNK):
        sl = slice(c * MOE_CHUNK, (c + 1) * MOE_CHUNK)
        x = r2_ref[sl, :]
        a = jnp.dot(x, w1_ref[0], preferred_element_type=jnp.float32)
        b = jnp.dot(x, w3_ref[0], preferred_element_type=jnp.float32)
        y = jnp.dot(a * jax.nn.sigmoid(a) * b, w2_ref[0],
                    preferred_element_type=jnp.float32)
        contrib = g[sl, :] * y

        @pl.when(e == 0)
        def _():
            out_ref[sl, :] = h2_ref[sl, :] + contrib

        @pl.when(e != 0)
        def _():
            out_ref[sl, :] += contrib


def _moe_call(r2, gates, h2, w1, w3, w2):
    return pl.pallas_call(
        _moe_body,
        grid=(E,),
        in_specs=[
            pl.BlockSpec((S, D), lambda e: (0, 0)),
            pl.BlockSpec((S, E), lambda e: (0, 0)),
            pl.BlockSpec((S, D), lambda e: (0, 0)),
            pl.BlockSpec((1, D, F), lambda e: (e, 0, 0)),
            pl.BlockSpec((1, D, F), lambda e: (e, 0, 0)),
            pl.BlockSpec((1, F, D), lambda e: (e, 0, 0)),
        ],
        out_specs=pl.BlockSpec((S, D), lambda e: (0, 0)),
        out_shape=jax.ShapeDtypeStruct((S, D), jnp.float32),
        compiler_params=pltpu.CompilerParams(
            vmem_limit_bytes=100 * 1024 * 1024),
    )(r2, gates, h2, w1, w3, w2)


def _final_body(h_ref, w_ref, out_ref):
    out_ref[...] = _rms(h_ref[...], w_ref[...])


def _final_call(h, w):
    return pl.pallas_call(
        _final_body,
        out_shape=jax.ShapeDtypeStruct((S, D), jnp.float32),
    )(h, w)


def _col_perm_q():
    import numpy as np
    n = np.arange(2 * QW)
    half, rest = n // QW, n % QW
    return (rest // HH) * HD + half * HH + rest % HH


def _col_perm_k():
    import numpy as np
    n = np.arange(2 * KW)
    half, rest = n // KW, n % KW
    return (rest // HH) * HD + half * HH + rest % HH


def _rope_tables():
    inv_freq = 1.0 / (THETA ** (jnp.arange(0, HD, 2).astype(jnp.float32) / HD))
    freqs = jnp.arange(S, dtype=jnp.float32)[:, None] * inv_freq[None, :]
    cosf, sinf = jnp.cos(freqs), jnp.sin(freqs)
    cq = jnp.tile(cosf, (1, 2 * QW // HH))
    sq = jnp.tile(sinf, (1, 2 * QW // HH))
    ck = jnp.tile(cosf, (1, 2 * KW // HH))
    sk = jnp.tile(sinf, (1, 2 * KW // HH))
    return cq, sq, ck, sk


@jax.jit
def _forward(x, params):
    cq, sq, ck, sk = _rope_tables()
    pq, pk = _col_perm_q(), _col_perm_k()
    h = x.reshape(S, D)
    for l in range(L):
        p = params['layer_%d' % l]
        q, k, v = _qkv_call(h, p['ln1'], p['wq'][:, pq], p['wk'][:, pk],
                            p['wv'], cq, sq, ck, sk)
        o = _attn_call(q, k, v)
        h2, r2, gates = _wo_router_call(h, o, p['wo'], p['ln2'], p['wg'])
        h = _moe_call(r2, gates, h2, p['w1'], p['w3'], p['w2'])
    return _final_call(h, params['final_ln']).reshape(1, S, D)


def kernel(input_ids, params):
    return _forward(input_ids, params)
